# baseline (speedup 1.0000x reference)
import math

import jax
import jax.numpy as jnp
import numpy as np
from jax import lax
from jax.experimental import pallas as pl
from jax.experimental.pallas import tpu as pltpu
from jax.experimental.pallas import tpu_sc as plsc

L_SP = 24
L_T = 24
FDIM = 2
LOG2_T = 20
TBL = 1 << LOG2_T
MASK = TBL - 1
SP_RES = np.floor(16.0 * ((4096.0 / 16.0) ** (np.arange(L_SP) / (L_SP - 1)))).astype(np.float32)
T_RES = np.floor(8.0 * ((8192.0 / 8.0) ** (np.arange(L_T) / (L_T - 1)))).astype(np.float32)
P2 = 2654435761
P3 = 805459861

NC = 2
NS = 16
NW = NC * NS
PG = 128
R_SP = L_SP * 8
R_T = L_T * 2
D_ENC = (L_SP + L_T) * FDIM
LVL_H = L_SP // 2
NSPH = LVL_H * 8 * FDIM * PG
NTC = R_T * FDIM * PG


def _prep_body(ct_ref, isp_ref, wsp_ref, it_ref, wt_ref):
    deg = math.pi / 180.0
    lat = ct_ref[0, :] * deg
    lon = ct_ref[1, :] * deg
    elev = ct_ref[2, :]
    t = ct_ref[3, :]
    R = 6371000.0
    r = R + elev
    s = 2.0 * (R + 10000.0)
    cl = jnp.cos(lat)
    x01 = r * cl * jnp.cos(lon) / s + 0.5
    y01 = r * cl * jnp.sin(lon) / s + 0.5
    z01 = r * jnp.sin(lat) / s + 0.5

    p2 = jnp.uint32(P2)
    p3 = jnp.uint32(P3)
    msk = jnp.uint32(MASK)
    one = jnp.uint32(1)

    isp_rows = []
    wsp_rows = []
    for l in range(L_SP):
        res = float(SP_RES[l])
        px = x01 * res
        py = y01 * res
        pz = z01 * res
        x0 = jnp.floor(px)
        y0 = jnp.floor(py)
        z0 = jnp.floor(pz)
        fx = px - x0
        fy = py - y0
        fz = pz - z0
        ix = x0.astype(jnp.int32).astype(jnp.uint32)
        iy = y0.astype(jnp.int32).astype(jnp.uint32)
        iz = z0.astype(jnp.int32).astype(jnp.uint32)
        hx = (ix, ix + one)
        hy = (iy * p2, (iy + one) * p2)
        hz = (iz * p3, (iz + one) * p3)
        wx = (1.0 - fx, fx)
        wy = (1.0 - fy, fy)
        wz = (1.0 - fz, fz)
        base = jnp.int32(l * TBL)
        for oi in (0, 1):
            for oj in (0, 1):
                for ok in (0, 1):
                    h = ((hx[oi] ^ hy[oj] ^ hz[ok]) & msk).astype(jnp.int32) + base
                    w = h * 2
                    isp_rows.append(w)
                    isp_rows.append(w + 1)
                    wsp_rows.append(wx[oi] * wy[oj] * wz[ok])

    it_rows = []
    wt_rows = []
    for l in range(L_T):
        res = float(T_RES[l])
        pt = t * res
        t0 = jnp.floor(pt)
        ft = pt - t0
        i0 = t0.astype(jnp.int32).astype(jnp.uint32)
        base = jnp.int32(l * TBL)
        h0 = (((i0 * p2) & msk).astype(jnp.int32) + base) * 2
        h1 = ((((i0 + one) * p2) & msk).astype(jnp.int32) + base) * 2
        it_rows.append(h0)
        it_rows.append(h0 + 1)
        it_rows.append(h1)
        it_rows.append(h1 + 1)
        wt_rows.append(1.0 - ft)
        wt_rows.append(ft)

    isp_ref[0] = jnp.stack(isp_rows, axis=0)
    wsp_ref[0] = jnp.stack(wsp_rows, axis=0)
    it_ref[0] = jnp.stack(it_rows, axis=0)
    wt_ref[0] = jnp.stack(wt_rows, axis=0)


def _mlp_body(f_ref, w1_ref, b1_ref, w2_ref, b2_ref, w3_ref, b3_ref, o_ref):
    f = f_ref[0]
    dn = (((0,), (0,)), ((), ()))
    h1 = lax.dot_general(w1_ref[...], f, dn, preferred_element_type=jnp.float32)
    h1 = jnp.maximum(h1 + b1_ref[...][:, None], 0.0)
    h2 = lax.dot_general(w2_ref[...], h1, dn, preferred_element_type=jnp.float32)
    h2 = jnp.maximum(h2 + b2_ref[...][:, None], 0.0)
    o = lax.dot_general(w3_ref[...], h2, dn, preferred_element_type=jnp.float32)
    o = o + b3_ref[...][:, None]
    o_ref[0] = 1.0 / (1.0 + jnp.exp(-o))


def _sc_body(isp_hbm, wsp_hbm, it_hbm, wt_hbm, tsp_hbm, tt_hbm, feats_hbm,
             isp_v, vsp_v, wsp_v, it_v, vt_v, wt_v, f_v, sem_sp, sem_t):
    wid = lax.axis_index("s") * NC + lax.axis_index("c")
    groups_per_tile = isp_hbm.shape[0] // NW

    def chunk(ci, carry):
        gidx = wid * groups_per_tile + ci
        pltpu.sync_copy(it_hbm.at[gidx], it_v)
        dt = pltpu.async_copy(tt_hbm.at[it_v], vt_v, sem_t)
        pltpu.sync_copy(wsp_hbm.at[gidx], wsp_v)
        pltpu.sync_copy(wt_hbm.at[gidx], wt_v)

        for h in (0, 1):
            pltpu.sync_copy(isp_hbm.at[gidx, pl.ds(h * NSPH, NSPH)], isp_v)
            pltpu.async_copy(tsp_hbm.at[isp_v], vsp_v, sem_sp).wait()

            def grp_sp(g, carry2, h=h):
                lane0 = g * 16

                def lvl_sp(ll, carry3, h=h):
                    l = h * LVL_H + ll
                    acc0 = jnp.zeros((16,), jnp.float32)
                    acc1 = jnp.zeros((16,), jnp.float32)
                    for c in range(8):
                        wv = wsp_v[l * 8 + c, pl.ds(lane0, 16)]
                        pos = ((ll * 8 + c) * 2) * PG + lane0
                        v0 = vsp_v[pl.ds(pos, 16)]
                        v1 = vsp_v[pl.ds(pos + PG, 16)]
                        acc0 = acc0 + v0 * wv
                        acc1 = acc1 + v1 * wv
                    f_v[2 * l, pl.ds(lane0, 16)] = acc0
                    f_v[2 * l + 1, pl.ds(lane0, 16)] = acc1
                    return carry3

                lax.fori_loop(0, LVL_H, lvl_sp, 0, unroll=False)
                return carry2

            lax.fori_loop(0, PG // 16, grp_sp, 0, unroll=False)

        dt.wait()

        def grp_t(g, carry2):
            lane0 = g * 16

            def lvl_t(l, carry3):
                w0 = wt_v[2 * l, pl.ds(lane0, 16)]
                w1 = wt_v[2 * l + 1, pl.ds(lane0, 16)]
                pos = (4 * l) * PG + lane0
                a0 = vt_v[pl.ds(pos, 16)] * w0 + vt_v[pl.ds(pos + 2 * PG, 16)] * w1
                a1 = vt_v[pl.ds(pos + PG, 16)] * w0 + vt_v[pl.ds(pos + 3 * PG, 16)] * w1
                f_v[2 * L_SP + 2 * l, pl.ds(lane0, 16)] = a0
                f_v[2 * L_SP + 2 * l + 1, pl.ds(lane0, 16)] = a1
                return carry3

            lax.fori_loop(0, L_T, lvl_t, 0, unroll=False)
            return carry2

        lax.fori_loop(0, PG // 16, grp_t, 0, unroll=False)

        pltpu.sync_copy(f_v, feats_hbm.at[gidx])
        return carry

    lax.fori_loop(0, groups_per_tile, chunk, 0, unroll=False)


def kernel(coords, spatial_table, temporal_table, W1, b1, W2, b2, W3, b3):
    B = coords.shape[0]
    G = B // PG
    assert B % (PG * NW) == 0

    coords_t = coords.T

    prep = pl.pallas_call(
        _prep_body,
        grid=(G,),
        in_specs=[pl.BlockSpec((4, PG), lambda i: (0, i))],
        out_specs=[
            pl.BlockSpec((1, 2 * R_SP, PG), lambda i: (i, 0, 0)),
            pl.BlockSpec((1, R_SP, PG), lambda i: (i, 0, 0)),
            pl.BlockSpec((1, 2 * R_T, PG), lambda i: (i, 0, 0)),
            pl.BlockSpec((1, R_T, PG), lambda i: (i, 0, 0)),
        ],
        out_shape=[
            jax.ShapeDtypeStruct((G, 2 * R_SP, PG), jnp.int32),
            jax.ShapeDtypeStruct((G, R_SP, PG), jnp.float32),
            jax.ShapeDtypeStruct((G, 2 * R_T, PG), jnp.int32),
            jax.ShapeDtypeStruct((G, R_T, PG), jnp.float32),
        ],
    )
    isp, wsp, it, wt = prep(coords_t)
    isp = isp.reshape(G, 2 * R_SP * PG)
    it = it.reshape(G, 2 * R_T * PG)

    tsp = spatial_table.reshape(L_SP * TBL * FDIM)
    tt = temporal_table.reshape(L_T * TBL * FDIM)

    mesh = plsc.VectorSubcoreMesh(core_axis_name="c", subcore_axis_name="s")
    enc = pl.kernel(
        _sc_body,
        out_type=jax.ShapeDtypeStruct((G, D_ENC, PG), jnp.float32),
        mesh=mesh,
        scratch_types=[
            pltpu.VMEM((NSPH,), jnp.int32),
            pltpu.VMEM((NSPH,), jnp.float32),
            pltpu.VMEM((R_SP, PG), jnp.float32),
            pltpu.VMEM((NTC,), jnp.int32),
            pltpu.VMEM((NTC,), jnp.float32),
            pltpu.VMEM((R_T, PG), jnp.float32),
            pltpu.VMEM((D_ENC, PG), jnp.float32),
            pltpu.SemaphoreType.DMA,
            pltpu.SemaphoreType.DMA,
        ],
        compiler_params=pltpu.CompilerParams(needs_layout_passes=False),
    )
    feats = enc(isp, wsp, it, wt, tsp, tt)

    mlp = pl.pallas_call(
        _mlp_body,
        grid=(G,),
        in_specs=[
            pl.BlockSpec((1, D_ENC, PG), lambda i: (i, 0, 0)),
            pl.BlockSpec((D_ENC, 256), lambda i: (0, 0)),
            pl.BlockSpec((256,), lambda i: (0,)),
            pl.BlockSpec((256, 256), lambda i: (0, 0)),
            pl.BlockSpec((256,), lambda i: (0,)),
            pl.BlockSpec((256, 3), lambda i: (0, 0)),
            pl.BlockSpec((3,), lambda i: (0,)),
        ],
        out_specs=pl.BlockSpec((1, 3, PG), lambda i: (i, 0, 0)),
        out_shape=jax.ShapeDtypeStruct((G, 3, PG), jnp.float32),
    )
    out3 = mlp(feats, W1, b1, W2, b2, W3, b3)
    return jnp.transpose(out3, (0, 2, 1)).reshape(B, 3)


# speedup vs baseline: 14.8069x; 14.8069x over previous
import math

import jax
import jax.numpy as jnp
import numpy as np
from jax import lax
from jax.experimental import pallas as pl
from jax.experimental.pallas import tpu as pltpu
from jax.experimental.pallas import tpu_sc as plsc

L_SP = 24
L_T = 24
FDIM = 2
LOG2_T = 20
TBL = 1 << LOG2_T
MASK = TBL - 1
SP_RES = np.floor(16.0 * ((4096.0 / 16.0) ** (np.arange(L_SP) / (L_SP - 1)))).astype(np.float32)
T_RES = np.floor(8.0 * ((8192.0 / 8.0) ** (np.arange(L_T) / (L_T - 1)))).astype(np.float32)
P2 = 2654435761
P3 = 805459861

NC = 2
NS = 16
NW = NC * NS
PG = 128
R_SP = L_SP * 8
R_T = L_T * 2
D_ENC = (L_SP + L_T) * FDIM
LVL_H = L_SP // 2
NSPH = LVL_H * 8 * FDIM * PG
NTC = R_T * FDIM * PG


def _prep_body(ct_ref, isp_ref, wsp_ref, it_ref, wt_ref):
    deg = math.pi / 180.0
    lat = ct_ref[0, :] * deg
    lon = ct_ref[1, :] * deg
    elev = ct_ref[2, :]
    t = ct_ref[3, :]
    R = 6371000.0
    r = R + elev
    s = 2.0 * (R + 10000.0)
    cl = jnp.cos(lat)
    x01 = r * cl * jnp.cos(lon) / s + 0.5
    y01 = r * cl * jnp.sin(lon) / s + 0.5
    z01 = r * jnp.sin(lat) / s + 0.5

    p2 = jnp.uint32(P2)
    p3 = jnp.uint32(P3)
    msk = jnp.uint32(MASK)
    one = jnp.uint32(1)

    isp_rows = []
    wsp_rows = []
    for l in range(L_SP):
        res = float(SP_RES[l])
        px = x01 * res
        py = y01 * res
        pz = z01 * res
        x0 = jnp.floor(px)
        y0 = jnp.floor(py)
        z0 = jnp.floor(pz)
        fx = px - x0
        fy = py - y0
        fz = pz - z0
        ix = x0.astype(jnp.int32).astype(jnp.uint32)
        iy = y0.astype(jnp.int32).astype(jnp.uint32)
        iz = z0.astype(jnp.int32).astype(jnp.uint32)
        hx = (ix, ix + one)
        hy = (iy * p2, (iy + one) * p2)
        hz = (iz * p3, (iz + one) * p3)
        wx = (1.0 - fx, fx)
        wy = (1.0 - fy, fy)
        wz = (1.0 - fz, fz)
        base = jnp.int32(l * (TBL * FDIM))
        for oi in (0, 1):
            for oj in (0, 1):
                for ok in (0, 1):
                    h = (hx[oi] ^ hy[oj] ^ hz[ok]) & msk
                    w = (((h >> 7) * 256) | (h & 127)).astype(jnp.int32) + base
                    isp_rows.append(w)
                    isp_rows.append(w + 128)
                    wsp_rows.append(wx[oi] * wy[oj] * wz[ok])

    it_rows = []
    wt_rows = []
    for l in range(L_T):
        res = float(T_RES[l])
        pt = t * res
        t0 = jnp.floor(pt)
        ft = pt - t0
        i0 = t0.astype(jnp.int32).astype(jnp.uint32)
        base = jnp.int32(l * (TBL * FDIM))
        h0 = (i0 * p2) & msk
        h1 = ((i0 + one) * p2) & msk
        w0 = (((h0 >> 7) * 256) | (h0 & 127)).astype(jnp.int32) + base
        w1 = (((h1 >> 7) * 256) | (h1 & 127)).astype(jnp.int32) + base
        it_rows.append(w0)
        it_rows.append(w0 + 128)
        it_rows.append(w1)
        it_rows.append(w1 + 128)
        wt_rows.append(1.0 - ft)
        wt_rows.append(ft)

    isp_ref[0] = jnp.stack(isp_rows, axis=0)
    wsp_ref[0] = jnp.stack(wsp_rows, axis=0)
    it_ref[0] = jnp.stack(it_rows, axis=0)
    wt_ref[0] = jnp.stack(wt_rows, axis=0)


def _mlp_body(f_ref, w1_ref, b1_ref, w2_ref, b2_ref, w3_ref, b3_ref, o_ref):
    f = f_ref[0]
    dn = (((0,), (0,)), ((), ()))
    h1 = lax.dot_general(w1_ref[...], f, dn, preferred_element_type=jnp.float32)
    h1 = jnp.maximum(h1 + b1_ref[...][:, None], 0.0)
    h2 = lax.dot_general(w2_ref[...], h1, dn, preferred_element_type=jnp.float32)
    h2 = jnp.maximum(h2 + b2_ref[...][:, None], 0.0)
    o = lax.dot_general(w3_ref[...], h2, dn, preferred_element_type=jnp.float32)
    o = o + b3_ref[...][:, None]
    o_ref[0] = 1.0 / (1.0 + jnp.exp(-o))


def _sc_body(isp_hbm, wsp_hbm, it_hbm, wt_hbm, tsp_hbm, tt_hbm, feats_hbm,
             isp_v, vsp_v, wsp_v, it_v, vt_v, wt_v, f_v, sem_sp, sem_t):
    wid = lax.axis_index("s") * NC + lax.axis_index("c")
    groups_per_tile = isp_hbm.shape[0] // NW

    def chunk(ci, carry):
        gidx = wid * groups_per_tile + ci
        pltpu.sync_copy(it_hbm.at[gidx], it_v)
        dt = pltpu.async_copy(tt_hbm.at[it_v], vt_v, sem_t)
        pltpu.sync_copy(wsp_hbm.at[gidx], wsp_v)
        pltpu.sync_copy(wt_hbm.at[gidx], wt_v)

        for h in (0, 1):
            pltpu.sync_copy(isp_hbm.at[gidx, pl.ds(h * NSPH, NSPH)], isp_v)
            pltpu.async_copy(tsp_hbm.at[isp_v], vsp_v, sem_sp).wait()

            def grp_sp(g, carry2, h=h):
                lane0 = g * 16

                def lvl_sp(ll, carry3, h=h):
                    l = h * LVL_H + ll
                    acc0 = jnp.zeros((16,), jnp.float32)
                    acc1 = jnp.zeros((16,), jnp.float32)
                    for c in range(8):
                        wv = wsp_v[l * 8 + c, pl.ds(lane0, 16)]
                        pos = ((ll * 8 + c) * 2) * PG + lane0
                        v0 = vsp_v[pl.ds(pos, 16)]
                        v1 = vsp_v[pl.ds(pos + PG, 16)]
                        acc0 = acc0 + v0 * wv
                        acc1 = acc1 + v1 * wv
                    f_v[2 * l, pl.ds(lane0, 16)] = acc0
                    f_v[2 * l + 1, pl.ds(lane0, 16)] = acc1
                    return carry3

                lax.fori_loop(0, LVL_H, lvl_sp, 0, unroll=False)
                return carry2

            lax.fori_loop(0, PG // 16, grp_sp, 0, unroll=False)

        dt.wait()

        def grp_t(g, carry2):
            lane0 = g * 16

            def lvl_t(l, carry3):
                w0 = wt_v[2 * l, pl.ds(lane0, 16)]
                w1 = wt_v[2 * l + 1, pl.ds(lane0, 16)]
                pos = (4 * l) * PG + lane0
                a0 = vt_v[pl.ds(pos, 16)] * w0 + vt_v[pl.ds(pos + 2 * PG, 16)] * w1
                a1 = vt_v[pl.ds(pos + PG, 16)] * w0 + vt_v[pl.ds(pos + 3 * PG, 16)] * w1
                f_v[2 * L_SP + 2 * l, pl.ds(lane0, 16)] = a0
                f_v[2 * L_SP + 2 * l + 1, pl.ds(lane0, 16)] = a1
                return carry3

            lax.fori_loop(0, L_T, lvl_t, 0, unroll=False)
            return carry2

        lax.fori_loop(0, PG // 16, grp_t, 0, unroll=False)

        pltpu.sync_copy(f_v, feats_hbm.at[gidx])
        return carry

    lax.fori_loop(0, groups_per_tile, chunk, 0, unroll=False)


def kernel(coords, spatial_table, temporal_table, W1, b1, W2, b2, W3, b3):
    B = coords.shape[0]
    G = B // PG
    assert B % (PG * NW) == 0

    coords_t = coords.T

    prep = pl.pallas_call(
        _prep_body,
        grid=(G,),
        in_specs=[pl.BlockSpec((4, PG), lambda i: (0, i))],
        out_specs=[
            pl.BlockSpec((1, 2 * R_SP, PG), lambda i: (i, 0, 0)),
            pl.BlockSpec((1, R_SP, PG), lambda i: (i, 0, 0)),
            pl.BlockSpec((1, 2 * R_T, PG), lambda i: (i, 0, 0)),
            pl.BlockSpec((1, R_T, PG), lambda i: (i, 0, 0)),
        ],
        out_shape=[
            jax.ShapeDtypeStruct((G, 2 * R_SP, PG), jnp.int32),
            jax.ShapeDtypeStruct((G, R_SP, PG), jnp.float32),
            jax.ShapeDtypeStruct((G, 2 * R_T, PG), jnp.int32),
            jax.ShapeDtypeStruct((G, R_T, PG), jnp.float32),
        ],
    )
    isp, wsp, it, wt = prep(coords_t)
    isp = isp.reshape(G, 2 * R_SP * PG)
    it = it.reshape(G, 2 * R_T * PG)

    tsp = (spatial_table.reshape(L_SP, TBL // 128, 128, FDIM)
           .transpose(0, 1, 3, 2).reshape(L_SP * TBL * FDIM))
    tt = (temporal_table.reshape(L_T, TBL // 128, 128, FDIM)
          .transpose(0, 1, 3, 2).reshape(L_T * TBL * FDIM))

    mesh = plsc.VectorSubcoreMesh(core_axis_name="c", subcore_axis_name="s")
    enc = pl.kernel(
        _sc_body,
        out_type=jax.ShapeDtypeStruct((G, D_ENC, PG), jnp.float32),
        mesh=mesh,
        scratch_types=[
            pltpu.VMEM((NSPH,), jnp.int32),
            pltpu.VMEM((NSPH,), jnp.float32),
            pltpu.VMEM((R_SP, PG), jnp.float32),
            pltpu.VMEM((NTC,), jnp.int32),
            pltpu.VMEM((NTC,), jnp.float32),
            pltpu.VMEM((R_T, PG), jnp.float32),
            pltpu.VMEM((D_ENC, PG), jnp.float32),
            pltpu.SemaphoreType.DMA,
            pltpu.SemaphoreType.DMA,
        ],
        compiler_params=pltpu.CompilerParams(needs_layout_passes=False),
    )
    feats = enc(isp, wsp, it, wt, tsp, tt)

    mlp = pl.pallas_call(
        _mlp_body,
        grid=(G,),
        in_specs=[
            pl.BlockSpec((1, D_ENC, PG), lambda i: (i, 0, 0)),
            pl.BlockSpec((D_ENC, 256), lambda i: (0, 0)),
            pl.BlockSpec((256,), lambda i: (0,)),
            pl.BlockSpec((256, 256), lambda i: (0, 0)),
            pl.BlockSpec((256,), lambda i: (0,)),
            pl.BlockSpec((256, 3), lambda i: (0, 0)),
            pl.BlockSpec((3,), lambda i: (0,)),
        ],
        out_specs=pl.BlockSpec((1, 3, PG), lambda i: (i, 0, 0)),
        out_shape=jax.ShapeDtypeStruct((G, 3, PG), jnp.float32),
    )
    out3 = mlp(feats, W1, b1, W2, b2, W3, b3)
    return jnp.transpose(out3, (0, 2, 1)).reshape(B, 3)


# speedup vs baseline: 16.6876x; 1.1270x over previous
import math

import jax
import jax.numpy as jnp
import numpy as np
from jax import lax
from jax.experimental import pallas as pl
from jax.experimental.pallas import tpu as pltpu
from jax.experimental.pallas import tpu_sc as plsc

L_SP = 24
L_T = 24
FDIM = 2
LOG2_T = 20
TBL = 1 << LOG2_T
MASK = TBL - 1
SP_RES = np.floor(16.0 * ((4096.0 / 16.0) ** (np.arange(L_SP) / (L_SP - 1)))).astype(np.float32)
T_RES = np.floor(8.0 * ((8192.0 / 8.0) ** (np.arange(L_T) / (L_T - 1)))).astype(np.float32)
P2 = 2654435761
P3 = 805459861

NC = 2
NS = 16
NW = NC * NS
PG = 128
LVL_Q = 6
NSPQ = LVL_Q * 8 * FDIM * PG
LVL_TH = 12
NTH = LVL_TH * 2 * FDIM * PG
D_ENC = (L_SP + L_T) * FDIM
TWORDS = TBL * FDIM


def _prep_body(ct_ref, o_ref):
    deg = math.pi / 180.0
    lat = ct_ref[0, :] * deg
    lon = ct_ref[1, :] * deg
    elev = ct_ref[2, :]
    t = ct_ref[3, :]
    R = 6371000.0
    r = R + elev
    s = 2.0 * (R + 10000.0)
    cl = jnp.cos(lat)
    x01 = r * cl * jnp.cos(lon) / s + 0.5
    y01 = r * cl * jnp.sin(lon) / s + 0.5
    z01 = r * jnp.sin(lat) / s + 0.5
    o_ref[0] = jnp.stack([x01, y01, z01, t], axis=0)


def _mlp_body(f_ref, w1_ref, b1_ref, w2_ref, b2_ref, w3_ref, b3_ref, o_ref):
    f = f_ref[0]
    dn = (((0,), (0,)), ((), ()))
    h1 = lax.dot_general(w1_ref[...], f, dn, preferred_element_type=jnp.float32)
    h1 = jnp.maximum(h1 + b1_ref[...][:, None], 0.0)
    h2 = lax.dot_general(w2_ref[...], h1, dn, preferred_element_type=jnp.float32)
    h2 = jnp.maximum(h2 + b2_ref[...][:, None], 0.0)
    o = lax.dot_general(w3_ref[...], h2, dn, preferred_element_type=jnp.float32)
    o = o + b3_ref[...][:, None]
    o_ref[0] = 1.0 / (1.0 + jnp.exp(-o))


def _phys_word(h):
    return (((h >> 7) * 256) | (h & 127)).astype(jnp.int32)


def _sc_body(xyzt_hbm, tsp_hbm, tt_hbm, feats_hbm,
             isp_v0, isp_v1, vsp_v0, vsp_v1, wsp_v, it_v, vt_v, wt_v,
             f_v, xyzt_v, res_sp_s, res_t_s, sem0, sem1, sem_t):
    wid = lax.axis_index("s") * NC + lax.axis_index("c")
    chunks_per_tile = xyzt_hbm.shape[0] // NW
    p2u = jnp.uint32(P2)
    p3u = jnp.uint32(P3)
    msku = jnp.uint32(MASK)

    for l in range(L_SP):
        res_sp_s[l] = jnp.float32(float(SP_RES[l]))
    for l in range(L_T):
        res_t_s[l] = jnp.float32(float(T_RES[l]))

    def hash_spatial_q(q, isp_v):
        def grp(g, carry):
            lane0 = g * 16
            x = xyzt_v[0, pl.ds(lane0, 16)]
            y = xyzt_v[1, pl.ds(lane0, 16)]
            z = xyzt_v[2, pl.ds(lane0, 16)]

            def lvl(ll, carry2):
                l = q * LVL_Q + ll
                res = res_sp_s[l]
                px = x * res
                py = y * res
                pz = z * res
                ix = px.astype(jnp.int32)
                iy = py.astype(jnp.int32)
                iz = pz.astype(jnp.int32)
                fx = px - ix.astype(jnp.float32)
                fy = py - iy.astype(jnp.float32)
                fz = pz - iz.astype(jnp.float32)
                ixu = ix.astype(jnp.uint32)
                iyu = iy.astype(jnp.uint32)
                izu = iz.astype(jnp.uint32)
                hx = (ixu, ixu + jnp.uint32(1))
                hy0 = iyu * p2u
                hy = (hy0, hy0 + p2u)
                hz0 = izu * p3u
                hz = (hz0, hz0 + p3u)
                wx = (1.0 - fx, fx)
                wy = (1.0 - fy, fy)
                wz = (1.0 - fz, fz)
                base = l * TWORDS
                for c in range(8):
                    oi, oj, ok = c >> 2, (c >> 1) & 1, c & 1
                    hh = (hx[oi] ^ hy[oj] ^ hz[ok]) & msku
                    w = _phys_word(hh) + base
                    pos = ((ll * 8 + c) * 2) * PG + lane0
                    isp_v[pl.ds(pos, 16)] = w
                    isp_v[pl.ds(pos + PG, 16)] = w + 128
                    wsp_v[q * 48 + ll * 8 + c, pl.ds(lane0, 16)] = \
                        wx[oi] * wy[oj] * wz[ok]
                return carry2

            lax.fori_loop(0, LVL_Q, lvl, 0, unroll=False)
            return carry

        lax.fori_loop(0, PG // 16, grp, 0, unroll=False)

    def hash_temporal(th):
        def grp(g, carry):
            lane0 = g * 16
            t = xyzt_v[3, pl.ds(lane0, 16)]

            def lvl(ll, carry2):
                l = th * LVL_TH + ll
                res = res_t_s[l]
                pt = t * res
                i0 = pt.astype(jnp.int32)
                ft = pt - i0.astype(jnp.float32)
                u = i0.astype(jnp.uint32) * p2u
                h0 = u & msku
                h1 = (u + p2u) & msku
                base = l * TWORDS
                w0 = _phys_word(h0) + base
                w1 = _phys_word(h1) + base
                pos = (4 * ll) * PG + lane0
                it_v[pl.ds(pos, 16)] = w0
                it_v[pl.ds(pos + PG, 16)] = w0 + 128
                it_v[pl.ds(pos + 2 * PG, 16)] = w1
                it_v[pl.ds(pos + 3 * PG, 16)] = w1 + 128
                wt_v[2 * l, pl.ds(lane0, 16)] = 1.0 - ft
                wt_v[2 * l + 1, pl.ds(lane0, 16)] = ft
                return carry2

            lax.fori_loop(0, LVL_TH, lvl, 0, unroll=False)
            return carry

        lax.fori_loop(0, PG // 16, grp, 0, unroll=False)

    def reduce_spatial_q(q, vsp_v):
        def grp(g, carry):
            lane0 = g * 16

            def lvl(ll, carry2):
                l = q * LVL_Q + ll
                acc0 = jnp.zeros((16,), jnp.float32)
                acc1 = jnp.zeros((16,), jnp.float32)
                for c in range(8):
                    wv = wsp_v[q * 48 + ll * 8 + c, pl.ds(lane0, 16)]
                    pos = ((ll * 8 + c) * 2) * PG + lane0
                    acc0 = acc0 + vsp_v[pl.ds(pos, 16)] * wv
                    acc1 = acc1 + vsp_v[pl.ds(pos + PG, 16)] * wv
                f_v[2 * l, pl.ds(lane0, 16)] = acc0
                f_v[2 * l + 1, pl.ds(lane0, 16)] = acc1
                return carry2

            lax.fori_loop(0, LVL_Q, lvl, 0, unroll=False)
            return carry

        lax.fori_loop(0, PG // 16, grp, 0, unroll=False)

    def reduce_temporal(th):
        def grp(g, carry):
            lane0 = g * 16

            def lvl(ll, carry2):
                l = th * LVL_TH + ll
                w0 = wt_v[2 * l, pl.ds(lane0, 16)]
                w1 = wt_v[2 * l + 1, pl.ds(lane0, 16)]
                pos = (4 * ll) * PG + lane0
                a0 = vt_v[pl.ds(pos, 16)] * w0 + vt_v[pl.ds(pos + 2 * PG, 16)] * w1
                a1 = vt_v[pl.ds(pos + PG, 16)] * w0 + vt_v[pl.ds(pos + 3 * PG, 16)] * w1
                f_v[2 * L_SP + 2 * l, pl.ds(lane0, 16)] = a0
                f_v[2 * L_SP + 2 * l + 1, pl.ds(lane0, 16)] = a1
                return carry2

            lax.fori_loop(0, LVL_TH, lvl, 0, unroll=False)
            return carry

        lax.fori_loop(0, PG // 16, grp, 0, unroll=False)

    def chunk(ci, carry):
        gidx = wid * chunks_per_tile + ci
        pltpu.sync_copy(xyzt_hbm.at[gidx], xyzt_v)

        hash_temporal(0)
        dt = pltpu.async_copy(tt_hbm.at[it_v], vt_v, sem_t)
        hash_spatial_q(0, isp_v0)
        d0 = pltpu.async_copy(tsp_hbm.at[isp_v0], vsp_v0, sem0)
        hash_spatial_q(1, isp_v1)
        d1 = pltpu.async_copy(tsp_hbm.at[isp_v1], vsp_v1, sem1)

        d0.wait()
        reduce_spatial_q(0, vsp_v0)
        hash_spatial_q(2, isp_v0)
        d0b = pltpu.async_copy(tsp_hbm.at[isp_v0], vsp_v0, sem0)

        d1.wait()
        reduce_spatial_q(1, vsp_v1)
        hash_spatial_q(3, isp_v1)
        d1b = pltpu.async_copy(tsp_hbm.at[isp_v1], vsp_v1, sem1)

        dt.wait()
        reduce_temporal(0)
        hash_temporal(1)
        dtb = pltpu.async_copy(tt_hbm.at[it_v], vt_v, sem_t)

        d0b.wait()
        reduce_spatial_q(2, vsp_v0)
        d1b.wait()
        reduce_spatial_q(3, vsp_v1)
        dtb.wait()
        reduce_temporal(1)

        pltpu.sync_copy(f_v, feats_hbm.at[gidx])
        return carry

    lax.fori_loop(0, chunks_per_tile, chunk, 0, unroll=False)


def kernel(coords, spatial_table, temporal_table, W1, b1, W2, b2, W3, b3):
    B = coords.shape[0]
    G = B // PG
    assert B % (PG * NW) == 0

    coords_t = coords.T

    prep = pl.pallas_call(
        _prep_body,
        grid=(G,),
        in_specs=[pl.BlockSpec((4, PG), lambda i: (0, i))],
        out_specs=pl.BlockSpec((1, 4, PG), lambda i: (i, 0, 0)),
        out_shape=jax.ShapeDtypeStruct((G, 4, PG), jnp.float32),
    )
    xyzt = prep(coords_t)

    tsp = (spatial_table.reshape(L_SP, TBL // 128, 128, FDIM)
           .transpose(0, 1, 3, 2).reshape(L_SP * TBL * FDIM))
    tt = (temporal_table.reshape(L_T, TBL // 128, 128, FDIM)
          .transpose(0, 1, 3, 2).reshape(L_T * TBL * FDIM))

    mesh = plsc.VectorSubcoreMesh(core_axis_name="c", subcore_axis_name="s")
    enc = pl.kernel(
        _sc_body,
        out_type=jax.ShapeDtypeStruct((G, D_ENC, PG), jnp.float32),
        mesh=mesh,
        scratch_types=[
            pltpu.VMEM((NSPQ,), jnp.int32),
            pltpu.VMEM((NSPQ,), jnp.int32),
            pltpu.VMEM((NSPQ,), jnp.float32),
            pltpu.VMEM((NSPQ,), jnp.float32),
            pltpu.VMEM((L_SP * 8, PG), jnp.float32),
            pltpu.VMEM((NTH,), jnp.int32),
            pltpu.VMEM((NTH,), jnp.float32),
            pltpu.VMEM((L_T * 2, PG), jnp.float32),
            pltpu.VMEM((D_ENC, PG), jnp.float32),
            pltpu.VMEM((4, PG), jnp.float32),
            pltpu.SMEM((L_SP,), jnp.float32),
            pltpu.SMEM((L_T,), jnp.float32),
            pltpu.SemaphoreType.DMA,
            pltpu.SemaphoreType.DMA,
            pltpu.SemaphoreType.DMA,
        ],
        compiler_params=pltpu.CompilerParams(needs_layout_passes=False),
    )
    feats = enc(xyzt, tsp, tt)

    mlp = pl.pallas_call(
        _mlp_body,
        grid=(G,),
        in_specs=[
            pl.BlockSpec((1, D_ENC, PG), lambda i: (i, 0, 0)),
            pl.BlockSpec((D_ENC, 256), lambda i: (0, 0)),
            pl.BlockSpec((256,), lambda i: (0,)),
            pl.BlockSpec((256, 256), lambda i: (0, 0)),
            pl.BlockSpec((256,), lambda i: (0,)),
            pl.BlockSpec((256, 3), lambda i: (0, 0)),
            pl.BlockSpec((3,), lambda i: (0,)),
        ],
        out_specs=pl.BlockSpec((1, 3, PG), lambda i: (i, 0, 0)),
        out_shape=jax.ShapeDtypeStruct((G, 3, PG), jnp.float32),
    )
    out3 = mlp(feats, W1, b1, W2, b2, W3, b3)
    return jnp.transpose(out3, (0, 2, 1)).reshape(B, 3)


# speedup vs baseline: 22.0780x; 1.3230x over previous
import math

import jax
import jax.numpy as jnp
import numpy as np
from jax import lax
from jax.experimental import pallas as pl
from jax.experimental.pallas import tpu as pltpu
from jax.experimental.pallas import tpu_sc as plsc

L_SP = 24
L_T = 24
FDIM = 2
LOG2_T = 20
TBL = 1 << LOG2_T
MASK = TBL - 1
SP_RES = np.floor(16.0 * ((4096.0 / 16.0) ** (np.arange(L_SP) / (L_SP - 1)))).astype(np.float32)
T_RES = np.floor(8.0 * ((8192.0 / 8.0) ** (np.arange(L_T) / (L_T - 1)))).astype(np.float32)
P2 = 2654435761
P3 = 805459861

NC = 2
NS = 16
NW = NC * NS
PG = 128
LVL_Q = 6
NSPQ = LVL_Q * 8 * FDIM * PG
LVL_TH = 12
NTH = LVL_TH * 2 * FDIM * PG
D_ENC = (L_SP + L_T) * FDIM
TWORDS = TBL * FDIM


def _prep_body(ct_ref, o_ref):
    deg = math.pi / 180.0
    lat = ct_ref[0, :] * deg
    lon = ct_ref[1, :] * deg
    elev = ct_ref[2, :]
    t = ct_ref[3, :]
    R = 6371000.0
    r = R + elev
    s = 2.0 * (R + 10000.0)
    cl = jnp.cos(lat)
    x01 = r * cl * jnp.cos(lon) / s + 0.5
    y01 = r * cl * jnp.sin(lon) / s + 0.5
    z01 = r * jnp.sin(lat) / s + 0.5
    o_ref[...] = jnp.stack([x01, y01, z01, t], axis=0)


def _mlp_body(f_ref, w1_ref, b1_ref, w2_ref, b2_ref, w3_ref, b3_ref, o_ref):
    f = f_ref[...]
    dn = (((0,), (0,)), ((), ()))
    h1 = lax.dot_general(w1_ref[...], f, dn, preferred_element_type=jnp.float32)
    h1 = jnp.maximum(h1 + b1_ref[...][:, None], 0.0)
    h2 = lax.dot_general(w2_ref[...], h1, dn, preferred_element_type=jnp.float32)
    h2 = jnp.maximum(h2 + b2_ref[...][:, None], 0.0)
    o = lax.dot_general(w3_ref[...], h2, dn, preferred_element_type=jnp.float32)
    o = o + b3_ref[...][:, None]
    o_ref[...] = 1.0 / (1.0 + jnp.exp(-o))


def _phys_word(h):
    return (((h >> 7) * 256) | (h & 127)).astype(jnp.int32)


def _sc_body(xyzt_hbm, tsp_hbm, tt_hbm, feats_hbm,
             isp_v0, isp_v1, vsp_v0, vsp_v1, wsp_v, it_v, vt_v, wt_v,
             f_v, xyzt_v, res_sp_s, res_t_s, sem0, sem1, sem_t):
    wid = lax.axis_index("s") * NC + lax.axis_index("c")
    chunks_per_tile = xyzt_hbm.shape[1] // (NW * PG)
    p2u = jnp.uint32(P2)
    p3u = jnp.uint32(P3)
    msku = jnp.uint32(MASK)

    for l in range(L_SP):
        res_sp_s[l] = jnp.float32(float(SP_RES[l]))
    for l in range(L_T):
        res_t_s[l] = jnp.float32(float(T_RES[l]))

    def hash_spatial_q(q, isp_v):
        def grp(g, carry):
            lane0 = g * 16
            x = xyzt_v[0, pl.ds(lane0, 16)]
            y = xyzt_v[1, pl.ds(lane0, 16)]
            z = xyzt_v[2, pl.ds(lane0, 16)]

            def lvl(ll, carry2):
                l = q * LVL_Q + ll
                res = res_sp_s[l]
                px = x * res
                py = y * res
                pz = z * res
                ix = px.astype(jnp.int32)
                iy = py.astype(jnp.int32)
                iz = pz.astype(jnp.int32)
                fx = px - ix.astype(jnp.float32)
                fy = py - iy.astype(jnp.float32)
                fz = pz - iz.astype(jnp.float32)
                ixu = ix.astype(jnp.uint32)
                iyu = iy.astype(jnp.uint32)
                izu = iz.astype(jnp.uint32)
                hx = (ixu, ixu + jnp.uint32(1))
                hy0 = iyu * p2u
                hy = (hy0, hy0 + p2u)
                hz0 = izu * p3u
                hz = (hz0, hz0 + p3u)
                wx = (1.0 - fx, fx)
                wy = (1.0 - fy, fy)
                wz = (1.0 - fz, fz)
                base = l * TWORDS
                for c in range(8):
                    oi, oj, ok = c >> 2, (c >> 1) & 1, c & 1
                    hh = (hx[oi] ^ hy[oj] ^ hz[ok]) & msku
                    w = _phys_word(hh) + base
                    pos = ((ll * 8 + c) * 2) * PG + lane0
                    isp_v[pl.ds(pos, 16)] = w
                    isp_v[pl.ds(pos + PG, 16)] = w + 128
                    wsp_v[q * 48 + ll * 8 + c, pl.ds(lane0, 16)] = \
                        wx[oi] * wy[oj] * wz[ok]
                return carry2

            lax.fori_loop(0, LVL_Q, lvl, 0, unroll=False)
            return carry

        lax.fori_loop(0, PG // 16, grp, 0, unroll=False)

    def hash_temporal(th):
        def grp(g, carry):
            lane0 = g * 16
            t = xyzt_v[3, pl.ds(lane0, 16)]

            def lvl(ll, carry2):
                l = th * LVL_TH + ll
                res = res_t_s[l]
                pt = t * res
                i0 = pt.astype(jnp.int32)
                ft = pt - i0.astype(jnp.float32)
                u = i0.astype(jnp.uint32) * p2u
                h0 = u & msku
                h1 = (u + p2u) & msku
                base = l * TWORDS
                w0 = _phys_word(h0) + base
                w1 = _phys_word(h1) + base
                pos = (4 * ll) * PG + lane0
                it_v[pl.ds(pos, 16)] = w0
                it_v[pl.ds(pos + PG, 16)] = w0 + 128
                it_v[pl.ds(pos + 2 * PG, 16)] = w1
                it_v[pl.ds(pos + 3 * PG, 16)] = w1 + 128
                wt_v[2 * l, pl.ds(lane0, 16)] = 1.0 - ft
                wt_v[2 * l + 1, pl.ds(lane0, 16)] = ft
                return carry2

            lax.fori_loop(0, LVL_TH, lvl, 0, unroll=False)
            return carry

        lax.fori_loop(0, PG // 16, grp, 0, unroll=False)

    def reduce_spatial_q(q, vsp_v):
        def grp(g, carry):
            lane0 = g * 16

            def lvl(ll, carry2):
                l = q * LVL_Q + ll
                acc0 = jnp.zeros((16,), jnp.float32)
                acc1 = jnp.zeros((16,), jnp.float32)
                for c in range(8):
                    wv = wsp_v[q * 48 + ll * 8 + c, pl.ds(lane0, 16)]
                    pos = ((ll * 8 + c) * 2) * PG + lane0
                    acc0 = acc0 + vsp_v[pl.ds(pos, 16)] * wv
                    acc1 = acc1 + vsp_v[pl.ds(pos + PG, 16)] * wv
                f_v[2 * l, pl.ds(lane0, 16)] = acc0
                f_v[2 * l + 1, pl.ds(lane0, 16)] = acc1
                return carry2

            lax.fori_loop(0, LVL_Q, lvl, 0, unroll=False)
            return carry

        lax.fori_loop(0, PG // 16, grp, 0, unroll=False)

    def reduce_temporal(th):
        def grp(g, carry):
            lane0 = g * 16

            def lvl(ll, carry2):
                l = th * LVL_TH + ll
                w0 = wt_v[2 * l, pl.ds(lane0, 16)]
                w1 = wt_v[2 * l + 1, pl.ds(lane0, 16)]
                pos = (4 * ll) * PG + lane0
                a0 = vt_v[pl.ds(pos, 16)] * w0 + vt_v[pl.ds(pos + 2 * PG, 16)] * w1
                a1 = vt_v[pl.ds(pos + PG, 16)] * w0 + vt_v[pl.ds(pos + 3 * PG, 16)] * w1
                f_v[2 * L_SP + 2 * l, pl.ds(lane0, 16)] = a0
                f_v[2 * L_SP + 2 * l + 1, pl.ds(lane0, 16)] = a1
                return carry2

            lax.fori_loop(0, LVL_TH, lvl, 0, unroll=False)
            return carry

        lax.fori_loop(0, PG // 16, grp, 0, unroll=False)

    def chunk(ci, carry):
        gidx = wid * chunks_per_tile + ci
        pbase = gidx * PG
        pltpu.sync_copy(xyzt_hbm.at[:, pl.ds(pbase, PG)], xyzt_v)

        hash_temporal(0)
        dt = pltpu.async_copy(tt_hbm.at[it_v], vt_v, sem_t)
        hash_spatial_q(0, isp_v0)
        d0 = pltpu.async_copy(tsp_hbm.at[isp_v0], vsp_v0, sem0)
        hash_spatial_q(1, isp_v1)
        d1 = pltpu.async_copy(tsp_hbm.at[isp_v1], vsp_v1, sem1)

        d0.wait()
        reduce_spatial_q(0, vsp_v0)
        hash_spatial_q(2, isp_v0)
        d0b = pltpu.async_copy(tsp_hbm.at[isp_v0], vsp_v0, sem0)

        d1.wait()
        reduce_spatial_q(1, vsp_v1)
        hash_spatial_q(3, isp_v1)
        d1b = pltpu.async_copy(tsp_hbm.at[isp_v1], vsp_v1, sem1)

        dt.wait()
        reduce_temporal(0)
        hash_temporal(1)
        dtb = pltpu.async_copy(tt_hbm.at[it_v], vt_v, sem_t)

        d0b.wait()
        reduce_spatial_q(2, vsp_v0)
        d1b.wait()
        reduce_spatial_q(3, vsp_v1)
        dtb.wait()
        reduce_temporal(1)

        pltpu.sync_copy(f_v, feats_hbm.at[:, pl.ds(pbase, PG)])
        return carry

    lax.fori_loop(0, chunks_per_tile, chunk, 0, unroll=False)


def kernel(coords, spatial_table, temporal_table, W1, b1, W2, b2, W3, b3):
    B = coords.shape[0]
    G = B // PG
    assert B % (PG * NW) == 0

    coords_t = coords.T

    prep = pl.pallas_call(
        _prep_body,
        grid=(B // 512,),
        in_specs=[pl.BlockSpec((4, 512), lambda i: (0, i))],
        out_specs=pl.BlockSpec((4, 512), lambda i: (0, i)),
        out_shape=jax.ShapeDtypeStruct((4, B), jnp.float32),
    )
    xyzt = prep(coords_t)

    tsp = (spatial_table.reshape(L_SP, TBL // 128, 128, FDIM)
           .transpose(0, 1, 3, 2).reshape(L_SP * TBL * FDIM))
    tt = (temporal_table.reshape(L_T, TBL // 128, 128, FDIM)
          .transpose(0, 1, 3, 2).reshape(L_T * TBL * FDIM))

    mesh = plsc.VectorSubcoreMesh(core_axis_name="c", subcore_axis_name="s")
    enc = pl.kernel(
        _sc_body,
        out_type=jax.ShapeDtypeStruct((D_ENC, B), jnp.float32),
        mesh=mesh,
        scratch_types=[
            pltpu.VMEM((NSPQ,), jnp.int32),
            pltpu.VMEM((NSPQ,), jnp.int32),
            pltpu.VMEM((NSPQ,), jnp.float32),
            pltpu.VMEM((NSPQ,), jnp.float32),
            pltpu.VMEM((L_SP * 8, PG), jnp.float32),
            pltpu.VMEM((NTH,), jnp.int32),
            pltpu.VMEM((NTH,), jnp.float32),
            pltpu.VMEM((L_T * 2, PG), jnp.float32),
            pltpu.VMEM((D_ENC, PG), jnp.float32),
            pltpu.VMEM((4, PG), jnp.float32),
            pltpu.SMEM((L_SP,), jnp.float32),
            pltpu.SMEM((L_T,), jnp.float32),
            pltpu.SemaphoreType.DMA,
            pltpu.SemaphoreType.DMA,
            pltpu.SemaphoreType.DMA,
        ],
        compiler_params=pltpu.CompilerParams(needs_layout_passes=False),
    )
    feats = enc(xyzt, tsp, tt)

    mlp = pl.pallas_call(
        _mlp_body,
        grid=(B // 512,),
        in_specs=[
            pl.BlockSpec((D_ENC, 512), lambda i: (0, i)),
            pl.BlockSpec((D_ENC, 256), lambda i: (0, 0)),
            pl.BlockSpec((256,), lambda i: (0,)),
            pl.BlockSpec((256, 256), lambda i: (0, 0)),
            pl.BlockSpec((256,), lambda i: (0,)),
            pl.BlockSpec((256, 3), lambda i: (0, 0)),
            pl.BlockSpec((3,), lambda i: (0,)),
        ],
        out_specs=pl.BlockSpec((3, 512), lambda i: (0, i)),
        out_shape=jax.ShapeDtypeStruct((3, B), jnp.float32),
    )
    out3 = mlp(feats, W1, b1, W2, b2, W3, b3)
    return out3.T


# speedup vs baseline: 25.8807x; 1.1722x over previous
import math

import jax
import jax.numpy as jnp
import numpy as np
from jax import lax
from jax.experimental import pallas as pl
from jax.experimental.pallas import tpu as pltpu
from jax.experimental.pallas import tpu_sc as plsc

L_SP = 24
L_T = 24
FDIM = 2
LOG2_T = 20
TBL = 1 << LOG2_T
MASK = TBL - 1
SP_RES = np.floor(16.0 * ((4096.0 / 16.0) ** (np.arange(L_SP) / (L_SP - 1)))).astype(np.float32)
T_RES = np.floor(8.0 * ((8192.0 / 8.0) ** (np.arange(L_T) / (L_T - 1)))).astype(np.float32)
P2 = 2654435761
P3 = 805459861

NC = 2
NS = 16
NW = NC * NS
PG = 128
LVL_Q = 6
NSPQ = LVL_Q * 8 * FDIM * PG
L_T_LUT = 21
L_T_HBM = L_T - L_T_LUT
NTH = L_T_HBM * 2 * FDIM * PG
T_E = [int(v) + 2 for v in T_RES[:L_T_LUT]]
T_REG = [-(-e // 16) * 16 for e in T_E]
T_WBASE = [0]
for _r in T_REG:
    T_WBASE.append(T_WBASE[-1] + 2 * _r)
LUT_WORDS = T_WBASE[-1]
D_ENC = (L_SP + L_T) * FDIM
TWORDS = TBL * FDIM


def _prep_body(ct_ref, o_ref):
    deg = math.pi / 180.0
    lat = ct_ref[0, :] * deg
    lon = ct_ref[1, :] * deg
    elev = ct_ref[2, :]
    t = ct_ref[3, :]
    R = 6371000.0
    r = R + elev
    s = 2.0 * (R + 10000.0)
    cl = jnp.cos(lat)
    x01 = r * cl * jnp.cos(lon) / s + 0.5
    y01 = r * cl * jnp.sin(lon) / s + 0.5
    z01 = r * jnp.sin(lat) / s + 0.5
    o_ref[...] = jnp.stack([x01, y01, z01, t], axis=0)


def _mlp_body(f_ref, w1_ref, b1_ref, w2_ref, b2_ref, w3_ref, b3_ref, o_ref):
    f = f_ref[...]
    dn = (((0,), (0,)), ((), ()))
    h1 = lax.dot_general(w1_ref[...], f, dn, preferred_element_type=jnp.float32)
    h1 = jnp.maximum(h1 + b1_ref[...][:, None], 0.0)
    h2 = lax.dot_general(w2_ref[...], h1, dn, preferred_element_type=jnp.float32)
    h2 = jnp.maximum(h2 + b2_ref[...][:, None], 0.0)
    o = lax.dot_general(w3_ref[...], h2, dn, preferred_element_type=jnp.float32)
    o = o + b3_ref[...][:, None]
    o_ref[...] = 1.0 / (1.0 + jnp.exp(-o))


def _phys_word(h):
    return (((h >> 7) * 256) | (h & 127)).astype(jnp.int32)


def _sc_body(xyzt_hbm, tsp_hbm, tt_hbm, feats_hbm,
             isp_v0, isp_v1, vsp_v0, vsp_v1, wsp_v, it_v, vt_v, wt_v,
             f_v, xyzt_v, lut_v, res_sp_s, res_t_s, lutw_s, sem0, sem1, sem_t):
    wid = lax.axis_index("s") * NC + lax.axis_index("c")
    chunks_per_tile = xyzt_hbm.shape[1] // (NW * PG)
    p2u = jnp.uint32(P2)
    p3u = jnp.uint32(P3)
    msku = jnp.uint32(MASK)

    for l in range(L_SP):
        res_sp_s[l] = jnp.float32(float(SP_RES[l]))
    for l in range(L_T):
        res_t_s[l] = jnp.float32(float(T_RES[l]))
    for l in range(L_T_LUT):
        lutw_s[l] = jnp.int32(T_WBASE[l])
    iota16 = lax.iota(jnp.int32, 16)

    for l in range(L_T_LUT):
        base = l * TWORDS

        def bgrp(k, carry, base=base):
            j = k * 16 + iota16
            u = j.astype(jnp.uint32) * p2u
            w = _phys_word(u & msku) + base
            jj = k * 32 + iota16 * 2
            plsc.store_scatter(isp_v0, [jj], w)
            plsc.store_scatter(isp_v0, [jj + 1], w + 128)
            return carry

        lax.fori_loop(0, T_REG[l] // 16, bgrp, 0, unroll=False)
        pltpu.async_copy(
            tt_hbm.at[isp_v0.at[pl.ds(0, 2 * T_REG[l])]],
            lut_v.at[pl.ds(T_WBASE[l], 2 * T_REG[l])], sem_t).wait()

    def hash_spatial_q(q, isp_v):
        def grp(g, carry):
            lane0 = g * 16
            x = xyzt_v[0, pl.ds(lane0, 16)]
            y = xyzt_v[1, pl.ds(lane0, 16)]
            z = xyzt_v[2, pl.ds(lane0, 16)]

            def lvl(ll, carry2):
                l = q * LVL_Q + ll
                res = res_sp_s[l]
                px = x * res
                py = y * res
                pz = z * res
                ix = px.astype(jnp.int32)
                iy = py.astype(jnp.int32)
                iz = pz.astype(jnp.int32)
                fx = px - ix.astype(jnp.float32)
                fy = py - iy.astype(jnp.float32)
                fz = pz - iz.astype(jnp.float32)
                ixu = ix.astype(jnp.uint32)
                iyu = iy.astype(jnp.uint32)
                izu = iz.astype(jnp.uint32)
                hx = (ixu, ixu + jnp.uint32(1))
                hy0 = iyu * p2u
                hy = (hy0, hy0 + p2u)
                hz0 = izu * p3u
                hz = (hz0, hz0 + p3u)
                wx = (1.0 - fx, fx)
                wy = (1.0 - fy, fy)
                wz = (1.0 - fz, fz)
                base = l * TWORDS
                for c in range(8):
                    oi, oj, ok = c >> 2, (c >> 1) & 1, c & 1
                    hh = (hx[oi] ^ hy[oj] ^ hz[ok]) & msku
                    w = _phys_word(hh) + base
                    pos = ((ll * 8 + c) * 2) * PG + lane0
                    isp_v[pl.ds(pos, 16)] = w
                    isp_v[pl.ds(pos + PG, 16)] = w + 128
                    wsp_v[q * 48 + ll * 8 + c, pl.ds(lane0, 16)] = \
                        wx[oi] * wy[oj] * wz[ok]
                return carry2

            lax.fori_loop(0, LVL_Q, lvl, 0, unroll=False)
            return carry

        lax.fori_loop(0, PG // 16, grp, 0, unroll=False)

    def hash_temporal():
        def grp(g, carry):
            lane0 = g * 16
            t = xyzt_v[3, pl.ds(lane0, 16)]

            def lvl(ll, carry2):
                l = L_T_LUT + ll
                res = res_t_s[l]
                pt = t * res
                i0 = pt.astype(jnp.int32)
                ft = pt - i0.astype(jnp.float32)
                u = i0.astype(jnp.uint32) * p2u
                h0 = u & msku
                h1 = (u + p2u) & msku
                base = l * TWORDS
                w0 = _phys_word(h0) + base
                w1 = _phys_word(h1) + base
                pos = (4 * ll) * PG + lane0
                it_v[pl.ds(pos, 16)] = w0
                it_v[pl.ds(pos + PG, 16)] = w0 + 128
                it_v[pl.ds(pos + 2 * PG, 16)] = w1
                it_v[pl.ds(pos + 3 * PG, 16)] = w1 + 128
                wt_v[2 * ll, pl.ds(lane0, 16)] = 1.0 - ft
                wt_v[2 * ll + 1, pl.ds(lane0, 16)] = ft
                return carry2

            lax.fori_loop(0, L_T_HBM, lvl, 0, unroll=False)
            return carry

        lax.fori_loop(0, PG // 16, grp, 0, unroll=False)

    def reduce_spatial_q(q, vsp_v):
        def grp(g, carry):
            lane0 = g * 16

            def lvl(ll, carry2):
                l = q * LVL_Q + ll
                acc0 = jnp.zeros((16,), jnp.float32)
                acc1 = jnp.zeros((16,), jnp.float32)
                for c in range(8):
                    wv = wsp_v[q * 48 + ll * 8 + c, pl.ds(lane0, 16)]
                    pos = ((ll * 8 + c) * 2) * PG + lane0
                    acc0 = acc0 + vsp_v[pl.ds(pos, 16)] * wv
                    acc1 = acc1 + vsp_v[pl.ds(pos + PG, 16)] * wv
                f_v[2 * l, pl.ds(lane0, 16)] = acc0
                f_v[2 * l + 1, pl.ds(lane0, 16)] = acc1
                return carry2

            lax.fori_loop(0, LVL_Q, lvl, 0, unroll=False)
            return carry

        lax.fori_loop(0, PG // 16, grp, 0, unroll=False)

    def reduce_temporal():
        def grp(g, carry):
            lane0 = g * 16

            def lvl(ll, carry2):
                l = L_T_LUT + ll
                w0 = wt_v[2 * ll, pl.ds(lane0, 16)]
                w1 = wt_v[2 * ll + 1, pl.ds(lane0, 16)]
                pos = (4 * ll) * PG + lane0
                a0 = vt_v[pl.ds(pos, 16)] * w0 + vt_v[pl.ds(pos + 2 * PG, 16)] * w1
                a1 = vt_v[pl.ds(pos + PG, 16)] * w0 + vt_v[pl.ds(pos + 3 * PG, 16)] * w1
                f_v[2 * L_SP + 2 * l, pl.ds(lane0, 16)] = a0
                f_v[2 * L_SP + 2 * l + 1, pl.ds(lane0, 16)] = a1
                return carry2

            lax.fori_loop(0, L_T_HBM, lvl, 0, unroll=False)
            return carry

        lax.fori_loop(0, PG // 16, grp, 0, unroll=False)

    def lut_temporal():
        def grp(g, carry):
            lane0 = g * 16
            t = xyzt_v[3, pl.ds(lane0, 16)]

            def lvl(l, carry2):
                res = res_t_s[l]
                wb = lutw_s[l]
                pt = t * res
                i0 = pt.astype(jnp.int32)
                ft = pt - i0.astype(jnp.float32)
                pp = wb + 2 * i0
                v00 = plsc.load_gather(lut_v, [pp])
                v01 = plsc.load_gather(lut_v, [pp + 1])
                v10 = plsc.load_gather(lut_v, [pp + 2])
                v11 = plsc.load_gather(lut_v, [pp + 3])
                w0 = 1.0 - ft
                f_v[2 * L_SP + 2 * l, pl.ds(lane0, 16)] = v00 * w0 + v10 * ft
                f_v[2 * L_SP + 2 * l + 1, pl.ds(lane0, 16)] = v01 * w0 + v11 * ft
                return carry2

            lax.fori_loop(0, L_T_LUT, lvl, 0, unroll=False)
            return carry

        lax.fori_loop(0, PG // 16, grp, 0, unroll=False)

    def chunk(ci, carry):
        gidx = wid * chunks_per_tile + ci
        pbase = gidx * PG
        pltpu.sync_copy(xyzt_hbm.at[:, pl.ds(pbase, PG)], xyzt_v)

        hash_temporal()
        dt = pltpu.async_copy(tt_hbm.at[it_v], vt_v, sem_t)
        hash_spatial_q(0, isp_v0)
        d0 = pltpu.async_copy(tsp_hbm.at[isp_v0], vsp_v0, sem0)
        hash_spatial_q(1, isp_v1)
        d1 = pltpu.async_copy(tsp_hbm.at[isp_v1], vsp_v1, sem1)

        d0.wait()
        reduce_spatial_q(0, vsp_v0)
        hash_spatial_q(2, isp_v0)
        d0b = pltpu.async_copy(tsp_hbm.at[isp_v0], vsp_v0, sem0)

        d1.wait()
        reduce_spatial_q(1, vsp_v1)
        hash_spatial_q(3, isp_v1)
        d1b = pltpu.async_copy(tsp_hbm.at[isp_v1], vsp_v1, sem1)

        lut_temporal()
        dt.wait()
        reduce_temporal()

        d0b.wait()
        reduce_spatial_q(2, vsp_v0)
        d1b.wait()
        reduce_spatial_q(3, vsp_v1)

        pltpu.sync_copy(f_v, feats_hbm.at[:, pl.ds(pbase, PG)])
        return carry

    lax.fori_loop(0, chunks_per_tile, chunk, 0, unroll=False)


def kernel(coords, spatial_table, temporal_table, W1, b1, W2, b2, W3, b3):
    B = coords.shape[0]
    G = B // PG
    assert B % (PG * NW) == 0

    coords_t = coords.T

    prep = pl.pallas_call(
        _prep_body,
        grid=(B // 512,),
        in_specs=[pl.BlockSpec((4, 512), lambda i: (0, i))],
        out_specs=pl.BlockSpec((4, 512), lambda i: (0, i)),
        out_shape=jax.ShapeDtypeStruct((4, B), jnp.float32),
    )
    xyzt = prep(coords_t)

    tsp = (spatial_table.reshape(L_SP, TBL // 128, 128, FDIM)
           .transpose(0, 1, 3, 2).reshape(L_SP * TBL * FDIM))
    tt = (temporal_table.reshape(L_T, TBL // 128, 128, FDIM)
          .transpose(0, 1, 3, 2).reshape(L_T * TBL * FDIM))

    mesh = plsc.VectorSubcoreMesh(core_axis_name="c", subcore_axis_name="s")
    enc = pl.kernel(
        _sc_body,
        out_type=jax.ShapeDtypeStruct((D_ENC, B), jnp.float32),
        mesh=mesh,
        scratch_types=[
            pltpu.VMEM((NSPQ,), jnp.int32),
            pltpu.VMEM((NSPQ,), jnp.int32),
            pltpu.VMEM((NSPQ,), jnp.float32),
            pltpu.VMEM((NSPQ,), jnp.float32),
            pltpu.VMEM((L_SP * 8, PG), jnp.float32),
            pltpu.VMEM((NTH,), jnp.int32),
            pltpu.VMEM((NTH,), jnp.float32),
            pltpu.VMEM((L_T_HBM * 2, PG), jnp.float32),
            pltpu.VMEM((D_ENC, PG), jnp.float32),
            pltpu.VMEM((4, PG), jnp.float32),
            pltpu.VMEM((LUT_WORDS,), jnp.float32),
            pltpu.SMEM((L_SP,), jnp.float32),
            pltpu.SMEM((L_T,), jnp.float32),
            pltpu.SMEM((L_T_LUT,), jnp.int32),
            pltpu.SemaphoreType.DMA,
            pltpu.SemaphoreType.DMA,
            pltpu.SemaphoreType.DMA,
        ],
        compiler_params=pltpu.CompilerParams(needs_layout_passes=False),
    )
    feats = enc(xyzt, tsp, tt)

    mlp = pl.pallas_call(
        _mlp_body,
        grid=(B // 512,),
        in_specs=[
            pl.BlockSpec((D_ENC, 512), lambda i: (0, i)),
            pl.BlockSpec((D_ENC, 256), lambda i: (0, 0)),
            pl.BlockSpec((256,), lambda i: (0,)),
            pl.BlockSpec((256, 256), lambda i: (0, 0)),
            pl.BlockSpec((256,), lambda i: (0,)),
            pl.BlockSpec((256, 3), lambda i: (0, 0)),
            pl.BlockSpec((3,), lambda i: (0,)),
        ],
        out_specs=pl.BlockSpec((3, 512), lambda i: (0, i)),
        out_shape=jax.ShapeDtypeStruct((3, B), jnp.float32),
    )
    out3 = mlp(feats, W1, b1, W2, b2, W3, b3)
    return out3.T


# speedup vs baseline: 27.0248x; 1.0442x over previous
import math

import jax
import jax.numpy as jnp
import numpy as np
from jax import lax
from jax.experimental import pallas as pl
from jax.experimental.pallas import tpu as pltpu
from jax.experimental.pallas import tpu_sc as plsc

L_SP = 24
L_T = 24
FDIM = 2
LOG2_T = 20
TBL = 1 << LOG2_T
MASK = TBL - 1
SP_RES = np.floor(16.0 * ((4096.0 / 16.0) ** (np.arange(L_SP) / (L_SP - 1)))).astype(np.float32)
T_RES = np.floor(8.0 * ((8192.0 / 8.0) ** (np.arange(L_T) / (L_T - 1)))).astype(np.float32)
P2 = 2654435761
P3 = 805459861

NC = 2
NS = 16
NW = NC * NS
PG = 128
LVL_Q = 6
NSPQ = LVL_Q * 8 * FDIM * PG
L_T_LUT = 21
L_T_HBM = L_T - L_T_LUT
NTH = L_T_HBM * 2 * FDIM * PG
T_E = [int(v) + 2 for v in T_RES[:L_T_LUT]]
T_REG = [-(-e // 16) * 16 for e in T_E]
T_WBASE = [0]
for _r in T_REG:
    T_WBASE.append(T_WBASE[-1] + 2 * _r)
LUT_WORDS = T_WBASE[-1]
D_ENC = (L_SP + L_T) * FDIM
TWORDS = TBL * FDIM


def _prep_body(ct_ref, o_ref):
    deg = math.pi / 180.0
    lat = ct_ref[0, :] * deg
    lon = ct_ref[1, :] * deg
    elev = ct_ref[2, :]
    t = ct_ref[3, :]
    R = 6371000.0
    r = R + elev
    s = 2.0 * (R + 10000.0)
    cl = jnp.cos(lat)
    x01 = r * cl * jnp.cos(lon) / s + 0.5
    y01 = r * cl * jnp.sin(lon) / s + 0.5
    z01 = r * jnp.sin(lat) / s + 0.5
    o_ref[...] = jnp.stack([x01, y01, z01, t], axis=0)


def _mlp_body(f_ref, w1_ref, b1_ref, w2_ref, b2_ref, w3_ref, b3_ref, o_ref):
    f = f_ref[...]
    dn = (((0,), (0,)), ((), ()))
    h1 = lax.dot_general(w1_ref[...], f, dn, preferred_element_type=jnp.float32)
    h1 = jnp.maximum(h1 + b1_ref[...][:, None], 0.0)
    h2 = lax.dot_general(w2_ref[...], h1, dn, preferred_element_type=jnp.float32)
    h2 = jnp.maximum(h2 + b2_ref[...][:, None], 0.0)
    o = lax.dot_general(w3_ref[...], h2, dn, preferred_element_type=jnp.float32)
    o = o + b3_ref[...][:, None]
    o_ref[...] = 1.0 / (1.0 + jnp.exp(-o))


def _phys_word(h):
    return (((h >> 7) * 256) | (h & 127)).astype(jnp.int32)


def _sc_body(xyzt_hbm, tsp_hbm, tt_hbm, feats_hbm,
             isp_v0, isp_v1, vsp_v0, vsp_v1, wsp_v, it_v, vt_v, wt_v,
             f_v, xyzt_v, lut_v, res_sp_s, res_t_s, lutw_s, sem0, sem1, sem_t):
    wid = lax.axis_index("s") * NC + lax.axis_index("c")
    chunks_per_tile = xyzt_hbm.shape[1] // (NW * PG)
    p2u = jnp.uint32(P2)
    p3u = jnp.uint32(P3)
    msku = jnp.uint32(MASK)

    for l in range(L_SP):
        res_sp_s[l] = jnp.float32(float(SP_RES[l]))
    for l in range(L_T):
        res_t_s[l] = jnp.float32(float(T_RES[l]))
    for l in range(L_T_LUT):
        lutw_s[l] = jnp.int32(T_WBASE[l])
    iota16 = lax.iota(jnp.int32, 16)

    for l in range(L_T_LUT):
        base = l * TWORDS

        def bgrp(k, carry, base=base):
            j = k * 16 + iota16
            u = j.astype(jnp.uint32) * p2u
            w = _phys_word(u & msku) + base
            jj = k * 32 + iota16 * 2
            plsc.store_scatter(isp_v0, [jj], w)
            plsc.store_scatter(isp_v0, [jj + 1], w + 128)
            return carry

        lax.fori_loop(0, T_REG[l] // 16, bgrp, 0, unroll=False)
        pltpu.async_copy(
            tt_hbm.at[isp_v0.at[pl.ds(0, 2 * T_REG[l])]],
            lut_v.at[pl.ds(T_WBASE[l], 2 * T_REG[l])], sem_t).wait()

    def hash_spatial_q(q, isp_v):
        def grp(g, carry):
            lane0 = g * 16
            x = xyzt_v[0, pl.ds(lane0, 16)]
            y = xyzt_v[1, pl.ds(lane0, 16)]
            z = xyzt_v[2, pl.ds(lane0, 16)]

            def lvl(ll, carry2):
                l = q * LVL_Q + ll
                res = res_sp_s[l]
                px = x * res
                py = y * res
                pz = z * res
                ix = px.astype(jnp.int32)
                iy = py.astype(jnp.int32)
                iz = pz.astype(jnp.int32)
                fx = px - ix.astype(jnp.float32)
                fy = py - iy.astype(jnp.float32)
                fz = pz - iz.astype(jnp.float32)
                ixu = ix.astype(jnp.uint32)
                iyu = iy.astype(jnp.uint32)
                izu = iz.astype(jnp.uint32)
                hx = (ixu, ixu + jnp.uint32(1))
                hy0 = iyu * p2u
                hy = (hy0, hy0 + p2u)
                hz0 = izu * p3u
                hz = (hz0, hz0 + p3u)
                wx = (1.0 - fx, fx)
                wy = (1.0 - fy, fy)
                wz = (1.0 - fz, fz)
                base = l * TWORDS
                for c in range(8):
                    oi, oj, ok = c >> 2, (c >> 1) & 1, c & 1
                    hh = (hx[oi] ^ hy[oj] ^ hz[ok]) & msku
                    w = _phys_word(hh) + base
                    pos = ((ll * 8 + c) * 2) * PG + lane0
                    isp_v[pl.ds(pos, 16)] = w
                    isp_v[pl.ds(pos + PG, 16)] = w + 128
                    wsp_v[q * 48 + ll * 8 + c, pl.ds(lane0, 16)] = \
                        wx[oi] * wy[oj] * wz[ok]
                return carry2

            lax.fori_loop(0, LVL_Q, lvl, 0, unroll=False)
            return carry

        lax.fori_loop(0, PG // 16, grp, 0, unroll=False)

    def hash_temporal():
        def grp(g, carry):
            lane0 = g * 16
            t = xyzt_v[3, pl.ds(lane0, 16)]

            def lvl(ll, carry2):
                l = L_T_LUT + ll
                res = res_t_s[l]
                pt = t * res
                i0 = pt.astype(jnp.int32)
                ft = pt - i0.astype(jnp.float32)
                u = i0.astype(jnp.uint32) * p2u
                h0 = u & msku
                h1 = (u + p2u) & msku
                base = l * TWORDS
                w0 = _phys_word(h0) + base
                w1 = _phys_word(h1) + base
                pos = (4 * ll) * PG + lane0
                it_v[pl.ds(pos, 16)] = w0
                it_v[pl.ds(pos + PG, 16)] = w0 + 128
                it_v[pl.ds(pos + 2 * PG, 16)] = w1
                it_v[pl.ds(pos + 3 * PG, 16)] = w1 + 128
                wt_v[2 * ll, pl.ds(lane0, 16)] = 1.0 - ft
                wt_v[2 * ll + 1, pl.ds(lane0, 16)] = ft
                return carry2

            lax.fori_loop(0, L_T_HBM, lvl, 0, unroll=False)
            return carry

        lax.fori_loop(0, PG // 16, grp, 0, unroll=False)

    def reduce_spatial_q(q, vsp_v):
        def grp(g, carry):
            lane0 = g * 16

            def lvl(ll, carry2):
                l = q * LVL_Q + ll
                acc0 = jnp.zeros((16,), jnp.float32)
                acc1 = jnp.zeros((16,), jnp.float32)
                for c in range(8):
                    wv = wsp_v[q * 48 + ll * 8 + c, pl.ds(lane0, 16)]
                    pos = ((ll * 8 + c) * 2) * PG + lane0
                    acc0 = acc0 + vsp_v[pl.ds(pos, 16)] * wv
                    acc1 = acc1 + vsp_v[pl.ds(pos + PG, 16)] * wv
                f_v[2 * l, pl.ds(lane0, 16)] = acc0
                f_v[2 * l + 1, pl.ds(lane0, 16)] = acc1
                return carry2

            lax.fori_loop(0, LVL_Q, lvl, 0, unroll=False)
            return carry

        lax.fori_loop(0, PG // 16, grp, 0, unroll=False)

    def reduce_temporal():
        def grp(g, carry):
            lane0 = g * 16

            def lvl(ll, carry2):
                l = L_T_LUT + ll
                w0 = wt_v[2 * ll, pl.ds(lane0, 16)]
                w1 = wt_v[2 * ll + 1, pl.ds(lane0, 16)]
                pos = (4 * ll) * PG + lane0
                a0 = vt_v[pl.ds(pos, 16)] * w0 + vt_v[pl.ds(pos + 2 * PG, 16)] * w1
                a1 = vt_v[pl.ds(pos + PG, 16)] * w0 + vt_v[pl.ds(pos + 3 * PG, 16)] * w1
                f_v[2 * L_SP + 2 * l, pl.ds(lane0, 16)] = a0
                f_v[2 * L_SP + 2 * l + 1, pl.ds(lane0, 16)] = a1
                return carry2

            lax.fori_loop(0, L_T_HBM, lvl, 0, unroll=False)
            return carry

        lax.fori_loop(0, PG // 16, grp, 0, unroll=False)

    def lut_temporal():
        def grp(g, carry):
            lane0 = g * 16
            t = xyzt_v[3, pl.ds(lane0, 16)]

            def lvl(l, carry2):
                res = res_t_s[l]
                wb = lutw_s[l]
                pt = t * res
                i0 = pt.astype(jnp.int32)
                ft = pt - i0.astype(jnp.float32)
                pp = wb + 2 * i0
                v00 = plsc.load_gather(lut_v, [pp])
                v01 = plsc.load_gather(lut_v, [pp + 1])
                v10 = plsc.load_gather(lut_v, [pp + 2])
                v11 = plsc.load_gather(lut_v, [pp + 3])
                w0 = 1.0 - ft
                f_v[2 * L_SP + 2 * l, pl.ds(lane0, 16)] = v00 * w0 + v10 * ft
                f_v[2 * L_SP + 2 * l + 1, pl.ds(lane0, 16)] = v01 * w0 + v11 * ft
                return carry2

            lax.fori_loop(0, L_T_LUT, lvl, 0, unroll=False)
            return carry

        lax.fori_loop(0, PG // 16, grp, 0, unroll=False)

    def chunk(ci, carry):
        gidx = wid * chunks_per_tile + ci
        pbase = gidx * PG
        pltpu.sync_copy(xyzt_hbm.at[:, pl.ds(pbase, PG)], xyzt_v)

        hash_temporal()
        dt = pltpu.async_copy(tt_hbm.at[it_v], vt_v, sem_t)
        hash_spatial_q(0, isp_v0)
        d0 = pltpu.async_copy(tsp_hbm.at[isp_v0], vsp_v0, sem0)
        hash_spatial_q(1, isp_v1)
        d1 = pltpu.async_copy(tsp_hbm.at[isp_v1], vsp_v1, sem1)

        d0.wait()
        reduce_spatial_q(0, vsp_v0)
        hash_spatial_q(2, isp_v0)
        d0b = pltpu.async_copy(tsp_hbm.at[isp_v0], vsp_v0, sem0)

        d1.wait()
        reduce_spatial_q(1, vsp_v1)
        hash_spatial_q(3, isp_v1)
        d1b = pltpu.async_copy(tsp_hbm.at[isp_v1], vsp_v1, sem1)

        lut_temporal()
        dt.wait()
        reduce_temporal()

        d0b.wait()
        reduce_spatial_q(2, vsp_v0)
        d1b.wait()
        reduce_spatial_q(3, vsp_v1)

        pltpu.sync_copy(f_v, feats_hbm.at[:, pl.ds(pbase, PG)])
        return carry

    lax.fori_loop(0, chunks_per_tile, chunk, 0, unroll=False)


def kernel(coords, spatial_table, temporal_table, W1, b1, W2, b2, W3, b3):
    B = coords.shape[0]
    BH = B // 2
    assert B % (2 * PG * NW) == 0

    coords_t = coords.T

    tsp = (spatial_table.reshape(L_SP, TBL // 128, 128, FDIM)
           .transpose(0, 1, 3, 2).reshape(L_SP * TBL * FDIM))
    tt = (temporal_table.reshape(L_T, TBL // 128, 128, FDIM)
          .transpose(0, 1, 3, 2).reshape(L_T * TBL * FDIM))

    prep = pl.pallas_call(
        _prep_body,
        grid=(BH // 512,),
        in_specs=[pl.BlockSpec((4, 512), lambda i: (0, i))],
        out_specs=pl.BlockSpec((4, 512), lambda i: (0, i)),
        out_shape=jax.ShapeDtypeStruct((4, BH), jnp.float32),
    )

    mesh = plsc.VectorSubcoreMesh(core_axis_name="c", subcore_axis_name="s")
    enc = pl.kernel(
        _sc_body,
        out_type=jax.ShapeDtypeStruct((D_ENC, BH), jnp.float32),
        mesh=mesh,
        scratch_types=[
            pltpu.VMEM((NSPQ,), jnp.int32),
            pltpu.VMEM((NSPQ,), jnp.int32),
            pltpu.VMEM((NSPQ,), jnp.float32),
            pltpu.VMEM((NSPQ,), jnp.float32),
            pltpu.VMEM((L_SP * 8, PG), jnp.float32),
            pltpu.VMEM((NTH,), jnp.int32),
            pltpu.VMEM((NTH,), jnp.float32),
            pltpu.VMEM((L_T_HBM * 2, PG), jnp.float32),
            pltpu.VMEM((D_ENC, PG), jnp.float32),
            pltpu.VMEM((4, PG), jnp.float32),
            pltpu.VMEM((LUT_WORDS,), jnp.float32),
            pltpu.SMEM((L_SP,), jnp.float32),
            pltpu.SMEM((L_T,), jnp.float32),
            pltpu.SMEM((L_T_LUT,), jnp.int32),
            pltpu.SemaphoreType.DMA,
            pltpu.SemaphoreType.DMA,
            pltpu.SemaphoreType.DMA,
        ],
        compiler_params=pltpu.CompilerParams(needs_layout_passes=False),
    )

    mlp = pl.pallas_call(
        _mlp_body,
        grid=(BH // 512,),
        in_specs=[
            pl.BlockSpec((D_ENC, 512), lambda i: (0, i)),
            pl.BlockSpec((D_ENC, 256), lambda i: (0, 0)),
            pl.BlockSpec((256,), lambda i: (0,)),
            pl.BlockSpec((256, 256), lambda i: (0, 0)),
            pl.BlockSpec((256,), lambda i: (0,)),
            pl.BlockSpec((256, 3), lambda i: (0, 0)),
            pl.BlockSpec((3,), lambda i: (0,)),
        ],
        out_specs=pl.BlockSpec((3, 512), lambda i: (0, i)),
        out_shape=jax.ShapeDtypeStruct((3, BH), jnp.float32),
    )

    outs = []
    for hb in range(2):
        xyzt = prep(lax.slice_in_dim(coords_t, hb * BH, (hb + 1) * BH, axis=1))
        feats = enc(xyzt, tsp, tt)
        outs.append(mlp(feats, W1, b1, W2, b2, W3, b3))
    return jnp.concatenate([o.T for o in outs], axis=0)
